# baseline (speedup 1.0000x reference)
import jax
import jax.numpy as jnp
from jax.experimental import pallas as pl
from jax.experimental.pallas import tpu as pltpu

_B, _L, _H, _E, _R, _TOP_K = 4, 2048, 1024, 8, 64, 2
_KR = _TOP_K * _R
_LT = 1024


def _router_body(cls_ref, rw_ref, idx_ref, gates_ref):
    logits = jax.lax.dot_general(
        cls_ref[...], rw_ref[...],
        (((1,), (1,)), ((), ())),
        preferred_element_type=jnp.float32,
        precision=jax.lax.Precision.HIGHEST,
    )
    eidx = jax.lax.broadcasted_iota(jnp.int32, (_B, _E), 1)
    m1 = jnp.max(logits, axis=1, keepdims=True)
    i1 = jnp.min(jnp.where(logits == m1, eidx, _E), axis=1, keepdims=True)
    rest = jnp.where(eidx == i1, -jnp.inf, logits)
    m2 = jnp.max(rest, axis=1, keepdims=True)
    i2 = jnp.min(jnp.where(rest == m2, eidx, _E), axis=1, keepdims=True)
    g1 = 1.0 / (1.0 + jnp.exp(m2 - m1))
    idx_ref[...] = jnp.concatenate([i1, i2], axis=1)
    gates_ref[...] = jnp.concatenate([g1, 1.0 - g1], axis=1)


def _mix_body(idx_ref, gates_ref, a0_ref, a1_ref, b0_ref, b1_ref,
              x_ref, o_ref):
    b = pl.program_id(0) // (_L // _LT)
    srow = jax.lax.broadcasted_iota(jnp.int32, (_TOP_K, _KR), 0)
    scol = jax.lax.broadcasted_iota(jnp.int32, (_TOP_K, _KR), 1)
    sel = (scol // _R == srow).astype(jnp.float32)
    gv = jnp.dot(gates_ref[pl.ds(b, 1), :], sel,
                 preferred_element_type=jnp.float32)

    a2 = jnp.concatenate([a0_ref[0], a1_ref[0]], axis=1)
    bcat = jnp.concatenate([b0_ref[0], b1_ref[0]], axis=0)
    xb = x_ref[0].astype(jnp.bfloat16)
    low = jnp.dot(xb, a2, preferred_element_type=jnp.float32)
    low = (low * gv).astype(jnp.bfloat16)
    up = jnp.dot(low, bcat, preferred_element_type=jnp.float32)
    o_ref[0] = x_ref[0] + up


def kernel(x, router_w, lora_a, lora_b):
    cls = x[:, 0, :]
    idx, gates = pl.pallas_call(
        _router_body,
        in_specs=[pl.BlockSpec((_B, _H), lambda: (0, 0)),
                  pl.BlockSpec((_E, _H), lambda: (0, 0))],
        out_specs=[pl.BlockSpec((_B, _TOP_K), lambda: (0, 0)),
                   pl.BlockSpec((_B, _TOP_K), lambda: (0, 0))],
        out_shape=[jax.ShapeDtypeStruct((_B, _TOP_K), jnp.int32),
                   jax.ShapeDtypeStruct((_B, _TOP_K), jnp.float32)],
    )(cls, router_w)

    a16 = lora_a.astype(jnp.bfloat16)
    b16 = lora_b.astype(jnp.bfloat16)
    grid_spec = pltpu.PrefetchScalarGridSpec(
        num_scalar_prefetch=1,
        grid=(_B * _L // _LT,),
        in_specs=[
            pl.BlockSpec((_B, _TOP_K), lambda t, i: (0, 0)),
            pl.BlockSpec((1, _H, _R),
                         lambda t, i: (i[t // (_L // _LT), 0], 0, 0)),
            pl.BlockSpec((1, _H, _R),
                         lambda t, i: (i[t // (_L // _LT), 1], 0, 0)),
            pl.BlockSpec((1, _R, _H),
                         lambda t, i: (i[t // (_L // _LT), 0], 0, 0)),
            pl.BlockSpec((1, _R, _H),
                         lambda t, i: (i[t // (_L // _LT), 1], 0, 0)),
            pl.BlockSpec((1, _LT, _H),
                         lambda t, i: (t // (_L // _LT), t % (_L // _LT), 0)),
        ],
        out_specs=pl.BlockSpec((1, _LT, _H),
                               lambda t, i: (t // (_L // _LT), t % (_L // _LT), 0)),
    )
    return pl.pallas_call(
        _mix_body,
        grid_spec=grid_spec,
        out_shape=jax.ShapeDtypeStruct((_B, _L, _H), jnp.float32),
    )(idx, gates, a16, a16, b16, b16, x)
